# baseline (speedup 1.0000x reference)
import jax
import jax.numpy as jnp
from jax.experimental import pallas as pl


_B_BLK = 512


def _ffn_kernel(x_ref, thi_ref, tlo_ref, w1_ref, b1_ref, w2_ref, b2_ref,
                out_ref):
    xi = x_ref[...]
    xb = xi.astype(jnp.bfloat16)
    s = jnp.dot(xb, thi_ref[...], preferred_element_type=jnp.float32)
    s += jnp.dot(xb, tlo_ref[...], preferred_element_type=jnp.float32)
    cnt = jnp.sum(xi, axis=1, keepdims=True).astype(jnp.float32)
    e = s / (cnt + 1e-6)
    h = jnp.maximum(
        jnp.dot(e, w1_ref[...], preferred_element_type=jnp.float32)
        + b1_ref[...],
        0.0,
    )
    out_ref[...] = (
        jnp.sum(h * w2_ref[...], axis=1, keepdims=True) + b2_ref[0, 0]
    )


def kernel(x, table, W1, b1, W2, b2):
    B, V = x.shape
    D = table.shape[1]
    H = W1.shape[1]
    thi = table.astype(jnp.bfloat16)
    tlo = (table - thi.astype(jnp.float32)).astype(jnp.bfloat16)
    b1r = b1.reshape(1, H)
    w2r = W2.reshape(1, H)
    b2r = b2.reshape(1, 1)
    grid = (B // _B_BLK,)
    out = pl.pallas_call(
        _ffn_kernel,
        grid=grid,
        in_specs=[
            pl.BlockSpec((_B_BLK, V), lambda i: (i, 0)),
            pl.BlockSpec((V, D), lambda i: (0, 0)),
            pl.BlockSpec((V, D), lambda i: (0, 0)),
            pl.BlockSpec((D, H), lambda i: (0, 0)),
            pl.BlockSpec((1, H), lambda i: (0, 0)),
            pl.BlockSpec((1, H), lambda i: (0, 0)),
            pl.BlockSpec((1, 1), lambda i: (0, 0)),
        ],
        out_specs=pl.BlockSpec((_B_BLK, 1), lambda i: (i, 0)),
        out_shape=jax.ShapeDtypeStruct((B, 1), jnp.float32),
    )(x, thi, tlo, W1, b1r, w2r, b2r)
    return out


# speedup vs baseline: 1.1612x; 1.1612x over previous
import jax
import jax.numpy as jnp
from jax.experimental import pallas as pl
from jax.experimental.pallas import tpu as pltpu


_B_BLK = 256


def _ffn_kernel(x_ref, table_ref, w1_ref, b1_ref, w2_ref, b2_ref, out_ref):
    xi = x_ref[...]
    xf = xi.astype(jnp.float32)
    s = jnp.dot(xf, table_ref[...], preferred_element_type=jnp.float32)
    cnt = jnp.sum(xf, axis=1, keepdims=True)
    e = s / (cnt + 1e-6)
    h = jnp.maximum(
        jnp.dot(e, w1_ref[...], preferred_element_type=jnp.float32)
        + b1_ref[...],
        0.0,
    )
    out_ref[...] = (
        jnp.sum(h * w2_ref[...], axis=1, keepdims=True) + b2_ref[0, 0]
    )


def kernel(x, table, W1, b1, W2, b2):
    B, V = x.shape
    D = table.shape[1]
    H = W1.shape[1]
    b1r = b1.reshape(1, H)
    w2r = W2.reshape(1, H)
    b2r = b2.reshape(1, 1)
    grid = (B // _B_BLK,)
    out = pl.pallas_call(
        _ffn_kernel,
        grid=grid,
        in_specs=[
            pl.BlockSpec((_B_BLK, V), lambda i: (i, 0)),
            pl.BlockSpec((V, D), lambda i: (0, 0)),
            pl.BlockSpec((D, H), lambda i: (0, 0)),
            pl.BlockSpec((1, H), lambda i: (0, 0)),
            pl.BlockSpec((1, H), lambda i: (0, 0)),
            pl.BlockSpec((1, 1), lambda i: (0, 0)),
        ],
        out_specs=pl.BlockSpec((_B_BLK, 1), lambda i: (i, 0)),
        out_shape=jax.ShapeDtypeStruct((B, 1), jnp.float32),
        compiler_params=pltpu.CompilerParams(
            dimension_semantics=("parallel",),
        ),
    )(x, table, W1, b1r, w2r, b2r)
    return out


# speedup vs baseline: 1.3694x; 1.1793x over previous
import jax
import jax.numpy as jnp
from jax.experimental import pallas as pl
from jax.experimental.pallas import tpu as pltpu


_B_BLK = 512


def _ffn_kernel(x_ref, table_ref, w1_ref, b1_ref, w2_ref, b2_ref, out_ref):
    xi = x_ref[...]
    xf = xi.astype(jnp.float32)
    s = jnp.dot(xf, table_ref[...], preferred_element_type=jnp.float32)
    cnt = jnp.sum(xf, axis=1, keepdims=True)
    e = s / (cnt + 1e-6)
    h = jnp.maximum(
        jnp.dot(e, w1_ref[...], preferred_element_type=jnp.float32)
        + b1_ref[...],
        0.0,
    )
    out_ref[...] = (
        jnp.sum(h * w2_ref[...], axis=1, keepdims=True) + b2_ref[0, 0]
    )


def kernel(x, table, W1, b1, W2, b2):
    B, V = x.shape
    D = table.shape[1]
    H = W1.shape[1]
    b1r = b1.reshape(1, H)
    w2r = W2.reshape(1, H)
    b2r = b2.reshape(1, 1)
    grid = (B // _B_BLK,)
    out = pl.pallas_call(
        _ffn_kernel,
        grid=grid,
        in_specs=[
            pl.BlockSpec((_B_BLK, V), lambda i: (i, 0)),
            pl.BlockSpec((V, D), lambda i: (0, 0)),
            pl.BlockSpec((D, H), lambda i: (0, 0)),
            pl.BlockSpec((1, H), lambda i: (0, 0)),
            pl.BlockSpec((1, H), lambda i: (0, 0)),
            pl.BlockSpec((1, 1), lambda i: (0, 0)),
        ],
        out_specs=pl.BlockSpec((_B_BLK, 1), lambda i: (i, 0)),
        out_shape=jax.ShapeDtypeStruct((B, 1), jnp.float32),
        compiler_params=pltpu.CompilerParams(
            dimension_semantics=("parallel",),
        ),
    )(x, table, W1, b1r, w2r, b2r)
    return out
